# baseline (speedup 1.0000x reference)
import functools

import jax
import jax.numpy as jnp
from jax import lax
from jax.experimental import pallas as pl
from jax.experimental.pallas import tpu as pltpu
from jax.experimental.pallas import tpu_sc as plsc

_L = 16
_NSLOT = 3


def _sc_kernel(d, rows_per_w, chunk_rows, x_hbm, pe_hbm, idx_hbm, out_hbm,
               idx_v, xbuf0, xbuf1, xbuf2, pebuf0, pebuf1, pebuf2,
               sx0, sx1, sx2, sp0, sp1, sp2, so0, so1, so2):
    cid = lax.axis_index("c")
    sid = lax.axis_index("s")
    wid = sid * 2 + cid
    base0 = wid * rows_per_w
    n_chunks = rows_per_w // chunk_rows
    slices_per_row = d // _L

    xbufs = (xbuf0, xbuf1, xbuf2)
    pebufs = (pebuf0, pebuf1, pebuf2)
    sxs = (sx0, sx1, sx2)
    sps = (sp0, sp1, sp2)
    sos = (so0, so1, so2)

    pltpu.sync_copy(idx_hbm.at[pl.ds(base0, rows_per_w)], idx_v)

    def issue_loads(c, slot):
        base = base0 + c * chunk_rows
        pltpu.make_async_copy(
            x_hbm.at[pl.ds(base, chunk_rows)], xbufs[slot], sxs[slot]).start()
        pltpu.make_async_copy(
            pe_hbm.at[idx_v.at[pl.ds(c * chunk_rows, chunk_rows)]],
            pebufs[slot], sps[slot]).start()

    def wait_loads(slot):
        pltpu.make_async_copy(
            x_hbm.at[pl.ds(0, chunk_rows)], xbufs[slot], sxs[slot]).wait()
        pltpu.make_async_copy(
            pe_hbm.at[idx_v.at[pl.ds(0, chunk_rows)]],
            pebufs[slot], sps[slot]).wait()

    def wait_store(slot):
        pltpu.make_async_copy(
            xbufs[slot], out_hbm.at[pl.ds(0, chunk_rows)], sos[slot]).wait()

    issue_loads(0, 0)
    issue_loads(1, 1)

    def process(c, slot, ahead_slot):
        @pl.when(c + 2 < n_chunks)
        def _():
            @pl.when(c >= 1)
            def _():
                wait_store(ahead_slot)
            issue_loads(c + 2, ahead_slot)

        wait_loads(slot)

        def add_row(r, _):
            @plsc.parallel_loop(0, slices_per_row, unroll=8)
            def _(ci):
                off = ci * _L
                plsc.addupdate(xbufs[slot].at[r, pl.ds(off, _L)],
                               pebufs[slot][r, pl.ds(off, _L)])
            return ()

        lax.fori_loop(0, chunk_rows, add_row, ())
        base = base0 + c * chunk_rows
        pltpu.make_async_copy(
            xbufs[slot], out_hbm.at[pl.ds(base, chunk_rows)],
            sos[slot]).start()

    def outer(g, _):
        for b in range(_NSLOT):
            process(_NSLOT * g + b, b, (b + 2) % _NSLOT)
        return ()

    n_main = (n_chunks - 1) // _NSLOT
    lax.fori_loop(0, n_main, outer, ())
    for c in range(n_main * _NSLOT, n_chunks):
        process(c, c % _NSLOT, (c + 2) % _NSLOT)
    for slot in range(_NSLOT):
        wait_store(slot)


def kernel(x, pe, pe_id):
    b, s, d = x.shape
    n_rows = b * s
    xf = x.reshape(n_rows, d)
    pef = pe.reshape(pe.shape[1], d)
    idxf = pe_id.reshape(n_rows).astype(jnp.int32)

    n_workers = 32
    rows_per_w = n_rows // n_workers
    chunk_rows = 16

    mesh = plsc.VectorSubcoreMesh(core_axis_name="c", subcore_axis_name="s",
                                  num_cores=2, num_subcores=16)
    run = pl.kernel(
        functools.partial(_sc_kernel, d, rows_per_w, chunk_rows),
        out_type=jax.ShapeDtypeStruct((n_rows, d), jnp.float32),
        mesh=mesh,
        scratch_types=(
            [pltpu.VMEM((rows_per_w,), jnp.int32)]
            + [pltpu.VMEM((chunk_rows, d), jnp.float32)] * 6
            + [pltpu.SemaphoreType.DMA] * 9
        ),
    )
    out = run(xf, pef, idxf)
    return out.reshape(b, s, d)
